# baseline (speedup 1.0000x reference)
import jax
import jax.numpy as jnp
from jax.experimental import pallas as pl
from jax.experimental.pallas import tpu as pltpu

_NT = (((1,), (1,)), ((), ()))


def _mlp_knots_kernel(x_ref, w1_ref, b1_ref, w2_ref, b2_ref, w3_ref, b3_ref,
                      out_ref):
    xw = x_ref[...]
    f32 = jnp.float32

    w1 = w1_ref[...]
    w1row = jnp.concatenate([w1] * 8, axis=1)
    w1tile = jnp.concatenate([w1row] * 8, axis=0)
    r1024 = jax.lax.broadcasted_iota(jnp.int32, (1024, 64), 0)
    c64 = jax.lax.broadcasted_iota(jnp.int32, (1024, 64), 1)
    w1big = jnp.where((r1024 // 128) == (c64 // 8), w1tile, 0.0)
    h = jax.lax.dot_general(xw, w1big, _NT, preferred_element_type=f32)
    b1big = jnp.tile(b1_ref[...], (1, 8))
    h = jnp.maximum(h + b1big, 0.0)

    k5 = jax.lax.broadcasted_iota(jnp.int32, (5, 96), 0)
    l96 = jax.lax.broadcasted_iota(jnp.int32, (5, 96), 1)
    blk = l96 // 12
    col = l96 % 12
    base_c = (((col >= 4) & (col < 8) & (k5 <= (col - 4))) | (col >= 8))

    acc_u = jnp.zeros((xw.shape[0], 96), f32)
    acc_s = jnp.zeros((xw.shape[0], 96), f32)
    for j in range(8):
        h1j = h[:, 128 * j:128 * (j + 1)]
        h2j = jax.lax.dot_general(h1j, w2_ref[...], _NT,
                                  preferred_element_type=f32)
        h2j = jnp.maximum(h2j + b2_ref[...], 0.0)
        lg = jax.lax.dot_general(h2j, w3_ref[...], _NT,
                                 preferred_element_type=f32)
        lg = lg + b3_ref[...]
        m = jnp.max(lg, axis=1, keepdims=True)
        e = jnp.exp(lg - m)
        cj = ((blk == j) & base_c).astype(f32)
        sj = (blk == j).astype(f32)
        acc_u = acc_u + jnp.dot(e, cj, preferred_element_type=f32)
        acc_s = acc_s + jnp.dot(e, sj, preferred_element_type=f32)

    out_ref[...] = acc_u * (1.0 / acc_s)


@jax.jit
def kernel(input, W1, b1, W2, b2, W3, b3):
    B = input.shape[1]
    g = B // 8
    xw = input.reshape(g, 64)
    out = pl.pallas_call(
        _mlp_knots_kernel,
        grid=(1,),
        in_specs=[
            pl.BlockSpec((g, 64), lambda i: (0, 0)),
            pl.BlockSpec((128, 8), lambda i: (0, 0)),
            pl.BlockSpec((1, 128), lambda i: (0, 0)),
            pl.BlockSpec((128, 128), lambda i: (0, 0)),
            pl.BlockSpec((1, 128), lambda i: (0, 0)),
            pl.BlockSpec((5, 128), lambda i: (0, 0)),
            pl.BlockSpec((1, 5), lambda i: (0, 0)),
        ],
        out_specs=pl.BlockSpec((g, 96), lambda i: (0, 0)),
        out_shape=jax.ShapeDtypeStruct((g, 96), jnp.float32),
        compiler_params=pltpu.CompilerParams(
            dimension_semantics=("arbitrary",),
        ),
    )(xw, W1, b1.reshape(1, -1), W2, b2.reshape(1, -1), W3,
      b3.reshape(1, -1))
    return out.reshape(B, 12)


# speedup vs baseline: 1.5951x; 1.5951x over previous
import jax
import jax.numpy as jnp
from jax.experimental import pallas as pl
from jax.experimental.pallas import tpu as pltpu

_BM = 2048

_NT = (((1,), (1,)), ((), ()))
_NN = (((1,), (0,)), ((), ()))


def _mlp_knots_kernel(x_ref, w1b_ref, w2b_ref, w3b_ref, out_ref):
    f32 = jnp.float32
    x = x_ref[0]
    bm = x.shape[0]
    xb = jnp.concatenate([x, jnp.ones((bm, 1), f32)], axis=1)

    h1 = jax.lax.dot_general(w1b_ref[...], xb, _NT,
                             preferred_element_type=f32)
    h1 = jnp.maximum(h1, 0.0)
    h1b = jnp.concatenate([h1, jnp.ones((1, bm), f32)], axis=0)

    h2 = jax.lax.dot_general(w2b_ref[...], h1b, _NN,
                             preferred_element_type=f32)
    h2 = jnp.maximum(h2, 0.0)
    h2b = jnp.concatenate([h2, jnp.ones((1, bm), f32)], axis=0)

    lg = jax.lax.dot_general(w3b_ref[...], h2b, _NN,
                             preferred_element_type=f32)
    m = jnp.max(lg, axis=0, keepdims=True)
    e = jnp.exp(lg - m)

    r12 = jax.lax.broadcasted_iota(jnp.int32, (12, 5), 0)
    k5 = jax.lax.broadcasted_iota(jnp.int32, (12, 5), 1)
    ct = (((r12 >= 4) & (r12 < 8) & (k5 <= (r12 - 4)))
          | (r12 >= 8)).astype(f32)

    u = jax.lax.dot_general(ct, e, _NN,
                            preferred_element_type=f32)
    out_ref[...] = u * (1.0 / u[8:9, :])


@jax.jit
def kernel(input, W1, b1, W2, b2, W3, b3):
    B = input.shape[1]
    f32 = jnp.float32
    w1b = jnp.concatenate([W1, b1.reshape(-1, 1)], axis=1)
    w2b = jnp.concatenate([W2, b2.reshape(-1, 1)], axis=1)
    w3b = jnp.concatenate([W3, b3.reshape(-1, 1)], axis=1)

    out = pl.pallas_call(
        _mlp_knots_kernel,
        grid=(B // _BM,),
        in_specs=[
            pl.BlockSpec((1, _BM, 8), lambda i: (0, i, 0)),
            pl.BlockSpec((128, 9), lambda i: (0, 0)),
            pl.BlockSpec((128, 129), lambda i: (0, 0)),
            pl.BlockSpec((5, 129), lambda i: (0, 0)),
        ],
        out_specs=pl.BlockSpec((12, _BM), lambda i: (0, i)),
        out_shape=jax.ShapeDtypeStruct((12, B), f32),
        compiler_params=pltpu.CompilerParams(
            dimension_semantics=("parallel",),
        ),
    )(input, w1b, w2b, w3b)
    return out.T
